# baseline (device time: 99831 ns/iter reference)
import jax
import jax.numpy as jnp
from jax import lax
from jax.experimental import pallas as pl
from jax.experimental.pallas import tpu as pltpu

N_DEV = 16
N_WBUF = 2

_OFFSETS = sorted(range(1, N_DEV), key=lambda t: -min(t, N_DEV - t)) + [0]


def kernel(x, w_mat):
    m_per, k = x.shape
    _, n = w_mat.shape
    n_per = n // N_DEV
    m_tot = m_per * N_DEV

    def body(x_ref, w_hbm, out_ref,
             w_buf, y_buf, recv_buf,
             copy_sems, send_sems, recv_sems):
        me = lax.axis_index("i")

        barrier = pltpu.get_barrier_semaphore()
        for t in range(1, N_DEV):
            pl.semaphore_signal(
                barrier, inc=1,
                device_id=((me + t) % N_DEV,),
                device_id_type=pl.DeviceIdType.MESH,
            )
        pl.semaphore_wait(barrier, N_DEV - 1)

        def col(step):
            return (me + _OFFSETS[step]) % N_DEV

        def start_fetch(step):
            cp = pltpu.make_async_copy(
                w_hbm.at[:, pl.ds(col(step) * n_per, n_per)],
                w_buf.at[step % N_WBUF],
                copy_sems.at[step % N_WBUF],
            )
            cp.start()
            return cp

        fetches = {s: start_fetch(s) for s in range(N_WBUF - 1)}

        sends = []
        for step in range(N_DEV):
            nxt = step + N_WBUF - 1
            if nxt < N_DEV:
                fetches[nxt] = start_fetch(nxt)
            fetches[step].wait()
            y = jnp.dot(
                x_ref[:, :], w_buf[step % N_WBUF],
                preferred_element_type=jnp.float32,
            )
            y = (y * jax.nn.sigmoid(y)).astype(jnp.bfloat16)
            if _OFFSETS[step] == 0:
                recv_buf[me, :, :] = y
            else:
                y_buf[step, :, :] = y
                rdma = pltpu.make_async_remote_copy(
                    src_ref=y_buf.at[step],
                    dst_ref=recv_buf.at[me],
                    send_sem=send_sems.at[step],
                    recv_sem=recv_sems.at[me],
                    device_id=(col(step),),
                    device_id_type=pl.DeviceIdType.MESH,
                )
                rdma.start()
                sends.append(rdma)

        for rdma in sends:
            rdma.wait_send()

        for off in _OFFSETS[:-1]:
            src_dev = (me - off) % N_DEV
            recv_wait = pltpu.make_async_remote_copy(
                src_ref=y_buf.at[0],
                dst_ref=recv_buf.at[src_dev],
                send_sem=send_sems.at[0],
                recv_sem=recv_sems.at[src_dev],
                device_id=(me,),
                device_id_type=pl.DeviceIdType.MESH,
            )
            recv_wait.wait_recv()

        out_ref[:, :] = (
            recv_buf[:, :, :].reshape(m_tot, n_per).astype(jnp.float32)
        )

    return pl.pallas_call(
        body,
        out_shape=jax.ShapeDtypeStruct((m_tot, n_per), jnp.float32),
        in_specs=[
            pl.BlockSpec(memory_space=pltpu.VMEM),
            pl.BlockSpec(memory_space=pl.ANY),
        ],
        out_specs=pl.BlockSpec(memory_space=pltpu.VMEM),
        scratch_shapes=[
            pltpu.VMEM((N_WBUF, k, n_per), jnp.float32),
            pltpu.VMEM((N_DEV, m_per, n_per), jnp.bfloat16),
            pltpu.VMEM((N_DEV, m_per, n_per), jnp.bfloat16),
            pltpu.SemaphoreType.DMA((N_WBUF,)),
            pltpu.SemaphoreType.DMA((N_DEV,)),
            pltpu.SemaphoreType.DMA((N_DEV,)),
        ],
        compiler_params=pltpu.CompilerParams(
            collective_id=0,
            vmem_limit_bytes=100 * 1024 * 1024,
        ),
    )(x, w_mat)


# device time: 96075 ns/iter; 1.0391x vs baseline; 1.0391x over previous
import jax
import jax.numpy as jnp
from jax import lax
from jax.experimental import pallas as pl
from jax.experimental.pallas import tpu as pltpu

N_DEV = 16
N_WBUF = 3

_OFFSETS = [8, 7, 9, 3, 6, 10, 5, 13, 11, 4, 12, 2, 14, 1, 15, 0]


def kernel(x, w_mat):
    m_per, k = x.shape
    _, n = w_mat.shape
    n_per = n // N_DEV
    m_tot = m_per * N_DEV

    def body(x_ref, w_hbm, out_ref,
             w_buf, y_buf, recv_buf,
             copy_sems, send_sems, recv_sems):
        me = lax.axis_index("i")

        def col(step):
            return (me + _OFFSETS[step]) % N_DEV

        def start_fetch(step):
            cp = pltpu.make_async_copy(
                w_hbm.at[:, pl.ds(col(step) * n_per, n_per)],
                w_buf.at[step % N_WBUF],
                copy_sems.at[step % N_WBUF],
            )
            cp.start()
            return cp

        fetches = {s: start_fetch(s) for s in range(N_WBUF - 1)}

        barrier = pltpu.get_barrier_semaphore()
        for t in range(1, N_DEV):
            pl.semaphore_signal(
                barrier, inc=1,
                device_id=((me + t) % N_DEV,),
                device_id_type=pl.DeviceIdType.MESH,
            )
        pl.semaphore_wait(barrier, N_DEV - 1)

        sends = []
        for step in range(N_DEV):
            nxt = step + N_WBUF - 1
            if nxt < N_DEV:
                fetches[nxt] = start_fetch(nxt)
            fetches[step].wait()
            y = jnp.dot(
                x_ref[:, :], w_buf[step % N_WBUF],
                preferred_element_type=jnp.float32,
            )
            y = y * jax.nn.sigmoid(y)
            if _OFFSETS[step] == 0:
                out_ref[pl.ds(me * m_per, m_per), :] = y
            else:
                y_buf[step, :, :] = y.astype(jnp.bfloat16)
                rdma = pltpu.make_async_remote_copy(
                    src_ref=y_buf.at[step],
                    dst_ref=recv_buf.at[me],
                    send_sem=send_sems.at[step],
                    recv_sem=recv_sems.at[me],
                    device_id=(col(step),),
                    device_id_type=pl.DeviceIdType.MESH,
                )
                rdma.start()
                sends.append(rdma)

        for off in _OFFSETS[:-1]:
            src_dev = (me - off) % N_DEV
            recv_wait = pltpu.make_async_remote_copy(
                src_ref=y_buf.at[0],
                dst_ref=recv_buf.at[src_dev],
                send_sem=send_sems.at[0],
                recv_sem=recv_sems.at[src_dev],
                device_id=(me,),
                device_id_type=pl.DeviceIdType.MESH,
            )
            recv_wait.wait_recv()
            out_ref[pl.ds(src_dev * m_per, m_per), :] = (
                recv_buf[src_dev, :, :].astype(jnp.float32)
            )

        for rdma in sends:
            rdma.wait_send()

    return pl.pallas_call(
        body,
        out_shape=jax.ShapeDtypeStruct((m_tot, n_per), jnp.float32),
        in_specs=[
            pl.BlockSpec(memory_space=pltpu.VMEM),
            pl.BlockSpec(memory_space=pl.ANY),
        ],
        out_specs=pl.BlockSpec(memory_space=pltpu.VMEM),
        scratch_shapes=[
            pltpu.VMEM((N_WBUF, k, n_per), jnp.float32),
            pltpu.VMEM((N_DEV, m_per, n_per), jnp.bfloat16),
            pltpu.VMEM((N_DEV, m_per, n_per), jnp.bfloat16),
            pltpu.SemaphoreType.DMA((N_WBUF,)),
            pltpu.SemaphoreType.DMA((N_DEV,)),
            pltpu.SemaphoreType.DMA((N_DEV,)),
        ],
        compiler_params=pltpu.CompilerParams(
            collective_id=0,
            vmem_limit_bytes=100 * 1024 * 1024,
        ),
    )(x, w_mat)


# device time: 94592 ns/iter; 1.0554x vs baseline; 1.0157x over previous
import jax
import jax.numpy as jnp
from jax import lax
from jax.experimental import pallas as pl
from jax.experimental.pallas import tpu as pltpu

N_DEV = 16
N_WBUF = 3

_OFFSETS = [8, 7, 9, 3, 6, 10, 5, 13, 11, 4, 12, 2, 14, 1, 15, 0]


def kernel(x, w_mat):
    m_per, k = x.shape
    _, n = w_mat.shape
    n_per = n // N_DEV
    m_tot = m_per * N_DEV

    def body(x_ref, w_hbm, out_ref,
             w_buf, y_buf, recv_buf,
             copy_sems, send_sems, recv_sems):
        me = lax.axis_index("i")

        def col(step):
            return (me + _OFFSETS[step]) % N_DEV

        def start_fetch(step):
            cp = pltpu.make_async_copy(
                w_hbm.at[:, pl.ds(col(step) * n_per, n_per)],
                w_buf.at[step % N_WBUF],
                copy_sems.at[step % N_WBUF],
            )
            cp.start()
            return cp

        fetches = {s: start_fetch(s) for s in range(N_WBUF - 1)}

        barrier = pltpu.get_barrier_semaphore()
        for t in range(1, N_DEV):
            pl.semaphore_signal(
                barrier, inc=1,
                device_id=((me + t) % N_DEV,),
                device_id_type=pl.DeviceIdType.MESH,
            )

        sends = []
        for step in range(N_DEV):
            nxt = step + N_WBUF - 1
            if nxt < N_DEV:
                fetches[nxt] = start_fetch(nxt)
            fetches[step].wait()
            y = jnp.dot(
                x_ref[:, :], w_buf[step % N_WBUF],
                preferred_element_type=jnp.float32,
            )
            y = y * jax.nn.sigmoid(y)
            if _OFFSETS[step] == 0:
                out_ref[pl.ds(me * m_per, m_per), :] = y
            else:
                y_buf[step, :, :] = y.astype(jnp.bfloat16)
                if not sends:
                    pl.semaphore_wait(barrier, N_DEV - 1)
                rdma = pltpu.make_async_remote_copy(
                    src_ref=y_buf.at[step],
                    dst_ref=recv_buf.at[me],
                    send_sem=send_sems.at[step],
                    recv_sem=recv_sems.at[me],
                    device_id=(col(step),),
                    device_id_type=pl.DeviceIdType.MESH,
                )
                rdma.start()
                sends.append(rdma)

        for off in _OFFSETS[:-1]:
            src_dev = (me - off) % N_DEV
            recv_wait = pltpu.make_async_remote_copy(
                src_ref=y_buf.at[0],
                dst_ref=recv_buf.at[src_dev],
                send_sem=send_sems.at[0],
                recv_sem=recv_sems.at[src_dev],
                device_id=(me,),
                device_id_type=pl.DeviceIdType.MESH,
            )
            recv_wait.wait_recv()
            out_ref[pl.ds(src_dev * m_per, m_per), :] = (
                recv_buf[src_dev, :, :].astype(jnp.float32)
            )

        for rdma in sends:
            rdma.wait_send()

    return pl.pallas_call(
        body,
        out_shape=jax.ShapeDtypeStruct((m_tot, n_per), jnp.float32),
        in_specs=[
            pl.BlockSpec(memory_space=pltpu.VMEM),
            pl.BlockSpec(memory_space=pl.ANY),
        ],
        out_specs=pl.BlockSpec(memory_space=pltpu.VMEM),
        scratch_shapes=[
            pltpu.VMEM((N_WBUF, k, n_per), jnp.float32),
            pltpu.VMEM((N_DEV, m_per, n_per), jnp.bfloat16),
            pltpu.VMEM((N_DEV, m_per, n_per), jnp.bfloat16),
            pltpu.SemaphoreType.DMA((N_WBUF,)),
            pltpu.SemaphoreType.DMA((N_DEV,)),
            pltpu.SemaphoreType.DMA((N_DEV,)),
        ],
        compiler_params=pltpu.CompilerParams(
            collective_id=0,
            vmem_limit_bytes=100 * 1024 * 1024,
        ),
    )(x, w_mat)
